# baseline (speedup 1.0000x reference)
import functools

import jax
import jax.numpy as jnp
from jax.experimental import pallas as pl
from jax.experimental.pallas import tpu as pltpu

IMG_SIZE = 512.0
NCHUNK = 8


def _head_kernel(*refs, tn, w_dim):
    x_refs = refs[:NCHUNK]
    w_ref, b_ref, lab_ref, box_ref, ce_ref = refs[NCHUNK:]
    w = w_ref[...].astype(jnp.bfloat16)
    b = b_ref[...]
    for k in range(NCHUNK):
        xb = x_refs[k][0].astype(jnp.bfloat16)
        acc = jnp.dot(w, xb, preferred_element_type=jnp.float32)
        acc = acc + b
        sl = pl.ds(k * tn, tn)
        lab_ref[0, :, sl] = acc[0:20]
        ce_ref[0, :, sl] = acc[24:25]
        d = jnp.exp(acc[20:24])
        hw = k * tn + jax.lax.broadcasted_iota(jnp.int32, (1, tn), 1)
        stride = IMG_SIZE / w_dim
        cy = ((hw // w_dim).astype(jnp.float32) + 0.5) * stride
        cx = ((hw % w_dim).astype(jnp.float32) + 0.5) * stride
        box_ref[0, :, sl] = jnp.concatenate(
            [cx - d[0:1], cy - d[1:2], cx + d[2:3], cy + d[3:4]], axis=0)


def kernel(x, Wc, bc, Wb, bb, Wce, bce):
    B, C, H, W = x.shape
    HW = H * W
    TN = HW // NCHUNK
    nclasses = Wc.shape[0]

    xf = x.reshape(B, C, HW)
    Wf = jnp.concatenate([Wc, Wb, Wce], axis=0)
    bf = jnp.concatenate([bc, bb, bce], axis=0)[:, None]

    def x_spec(k):
        return pl.BlockSpec((1, C, TN), lambda i, k=k: (i, 0, k))

    labels, boxes, ctr = pl.pallas_call(
        functools.partial(_head_kernel, tn=TN, w_dim=W),
        grid=(B,),
        in_specs=[x_spec(k) for k in range(NCHUNK)] + [
            pl.BlockSpec((nclasses + 5, C), lambda i: (0, 0)),
            pl.BlockSpec((nclasses + 5, 1), lambda i: (0, 0)),
        ],
        out_specs=[
            pl.BlockSpec((1, nclasses, HW), lambda i: (i, 0, 0)),
            pl.BlockSpec((1, 4, HW), lambda i: (i, 0, 0)),
            pl.BlockSpec((1, 1, HW), lambda i: (i, 0, 0)),
        ],
        out_shape=[
            jax.ShapeDtypeStruct((B, nclasses, HW), jnp.float32),
            jax.ShapeDtypeStruct((B, 4, HW), jnp.float32),
            jax.ShapeDtypeStruct((B, 1, HW), jnp.float32),
        ],
        compiler_params=pltpu.CompilerParams(
            dimension_semantics=("parallel",)),
    )(*([xf] * NCHUNK), Wf, bf)

    return (labels.reshape(B, nclasses, H, W),
            boxes.reshape(B, 4, H, W),
            ctr.reshape(B, 1, H, W))


# speedup vs baseline: 2.5193x; 2.5193x over previous
import functools

import jax
import jax.numpy as jnp
from jax.experimental import pallas as pl
from jax.experimental.pallas import tpu as pltpu

IMG_SIZE = 512.0


def _head_kernel(x_ref, w2_ref, b2_ref, lab_ref, box_ref, ce_ref, acc_ref,
                 *, h_dim, w_dim, nclasses):
    ntiles = h_dim // 8
    nk = w2_ref.shape[1] // 256
    no = nclasses + 5
    x = x_ref[0]
    w2 = w2_ref[...]
    stride = IMG_SIZE / w_dim

    for k in range(nk - 1):
        wk = w2[:, 256 * k:256 * (k + 1)]
        for t in range(ntiles):
            xk = x[32 * k:32 * (k + 1), 8 * t:8 * (t + 1), :]
            xk = xk.reshape(256, w_dim).astype(jnp.bfloat16)
            p = jnp.dot(wk, xk, preferred_element_type=jnp.float32)
            sl = pl.ds(w_dim * t, w_dim)
            if k == 0:
                acc_ref[:, sl] = p
            else:
                acc_ref[:, sl] += p

    k = nk - 1
    wk = w2[:, 256 * k:256 * (k + 1)]
    b2 = b2_ref[...]
    for t in range(ntiles):
        xk = x[32 * k:32 * (k + 1), 8 * t:8 * (t + 1), :]
        xk = xk.reshape(256, w_dim).astype(jnp.bfloat16)
        p = jnp.dot(wk, xk, preferred_element_type=jnp.float32)
        vals = acc_ref[:, pl.ds(w_dim * t, w_dim)] + p + b2
        hs = pl.ds(8 * t, 8)
        lab_ref[0, :, hs, :] = vals[0:8 * nclasses].reshape(nclasses, 8, w_dim)
        ce_ref[0, :, hs, :] = vals[8 * (no - 1):8 * no].reshape(1, 8, w_dim)
        d = jnp.exp(vals[8 * nclasses:8 * (nclasses + 4)].reshape(4, 8, w_dim))
        hh = 8 * t + jax.lax.broadcasted_iota(jnp.int32, (1, 8, w_dim), 1)
        cy = (hh.astype(jnp.float32) + 0.5) * stride
        cx = (jax.lax.broadcasted_iota(jnp.int32, (1, 8, w_dim), 2)
              .astype(jnp.float32) + 0.5) * stride
        box_ref[0, :, hs, :] = jnp.concatenate(
            [cx - d[0:1], cy - d[1:2], cx + d[2:3], cy + d[3:4]], axis=0)


def kernel(x, Wc, bc, Wb, bb, Wce, bce):
    B, C, H, W = x.shape
    nclasses = Wc.shape[0]
    no = nclasses + 5

    Wf = jnp.concatenate([Wc, Wb, Wce], axis=0)
    bf = jnp.concatenate([bc, bb, bce], axis=0)
    eye8 = jnp.eye(8, dtype=jnp.float32)
    W2 = (Wf[:, None, :, None] * eye8[None, :, None, :]
          ).reshape(8 * no, 8 * C).astype(jnp.bfloat16)
    b2 = jnp.repeat(bf, 8)[:, None]

    labels, boxes, ctr = pl.pallas_call(
        functools.partial(_head_kernel, h_dim=H, w_dim=W, nclasses=nclasses),
        grid=(B,),
        in_specs=[
            pl.BlockSpec((1, C, H, W), lambda i: (i, 0, 0, 0)),
            pl.BlockSpec((8 * no, 8 * C), lambda i: (0, 0)),
            pl.BlockSpec((8 * no, 1), lambda i: (0, 0)),
        ],
        out_specs=[
            pl.BlockSpec((1, nclasses, H, W), lambda i: (i, 0, 0, 0)),
            pl.BlockSpec((1, 4, H, W), lambda i: (i, 0, 0, 0)),
            pl.BlockSpec((1, 1, H, W), lambda i: (i, 0, 0, 0)),
        ],
        out_shape=[
            jax.ShapeDtypeStruct((B, nclasses, H, W), jnp.float32),
            jax.ShapeDtypeStruct((B, 4, H, W), jnp.float32),
            jax.ShapeDtypeStruct((B, 1, H, W), jnp.float32),
        ],
        scratch_shapes=[pltpu.VMEM((8 * no, (H // 8) * W), jnp.float32)],
        compiler_params=pltpu.CompilerParams(
            dimension_semantics=("parallel",)),
    )(x, W2, b2)

    return (labels, boxes, ctr)
